# Initial kernel scaffold; baseline (speedup 1.0000x reference)
import functools

import jax
import jax.numpy as jnp
from jax import lax
from jax.experimental import pallas as pl
from jax.experimental.pallas import tpu as pltpu
from jax.experimental.pallas import tpu_sc as plsc

N = 10000
E = 160000
D_IN = 256
D_HID = 512

NC = 2
NS = 16
L = 16

IDX_W = 128
NB = 80
EPT = NB * IDX_W
E_PAD1 = NS * EPT
N_PAD = 10016
ZROWS = N_PAD // NS
OROWS = N // NS

EPW = 5008
E_PAD2 = NC * NS * EPW

R = 2000
G = N // R

_MESH = dict(core_axis_name="c", subcore_axis_name="s")


def _segsum_body(table_hbm, agg_sh, src_v, dst_v, rb0, rb1, sem0,
                 sem1, chunk):
    tbl = table_hbm.at[chunk]

    def start(b, buf, sem):
        pltpu.async_copy(tbl.at[src_v.at[b]], buf, sem)

    def wait(buf, sem):
        pltpu.make_async_copy(tbl.at[src_v.at[0]], buf, sem).wait()

    start(0, rb0, sem0)
    start(1, rb1, sem1)

    def body(k, carry):
        b0 = 2 * k
        wait(rb0, sem0)

        @pl.when(k < NB // 2 - 1)
        def _():
            start(b0 + 2, rb0, sem0)

        pltpu.sync_copy(rb0, agg_sh.at[dst_v.at[b0]], add=True)
        wait(rb1, sem1)

        @pl.when(k < NB // 2 - 1)
        def _():
            start(b0 + 3, rb1, sem1)

        pltpu.sync_copy(rb1, agg_sh.at[dst_v.at[b0 + 1]], add=True)
        return carry

    lax.fori_loop(0, NB // 2, body, 0)


@functools.partial(
    pl.kernel,
    out_type=(
        jax.ShapeDtypeStruct((2, N, 128), jnp.float32),
        jax.ShapeDtypeStruct((NS, N), jnp.float32),
    ),
    mesh=plsc.VectorSubcoreMesh(**_MESH),
    scratch_types=(
        pltpu.VMEM((NB, IDX_W), jnp.int32),
        pltpu.VMEM((NB, IDX_W), jnp.int32),
        pltpu.VMEM((IDX_W, 128), jnp.float32),
        pltpu.VMEM((IDX_W, 128), jnp.float32),
        pltpu.VMEM((N_PAD,), jnp.float32),
        pltpu.VMEM_SHARED((N_PAD, 128), jnp.float32),
        pltpu.SemaphoreType.DMA,
        pltpu.SemaphoreType.DMA,
    ),
)
def _sc_agg1(xT_hbm, srcp_hbm, dstp_hbm, zeros_hbm, agg_out, cnt_out,
             src_v, dst_v, rb0, rb1, cnt_v, agg_sh, sem0, sem1):
    c = lax.axis_index("c")
    s = lax.axis_index("s")

    pltpu.sync_copy(srcp_hbm.at[s], src_v)
    pltpu.sync_copy(dstp_hbm.at[s], dst_v)
    pltpu.sync_copy(zeros_hbm.at[pl.ds(s * ZROWS, ZROWS)],
                    agg_sh.at[pl.ds(s * ZROWS, ZROWS)])

    @pl.when(c == 0)
    def _():
        def zbody(i, carry):
            cnt_v[pl.ds(i * L, L)] = jnp.zeros((L,), jnp.float32)
            return carry
        lax.fori_loop(0, N_PAD // L, zbody, 0)

        ones = jnp.ones((L,), jnp.float32)

        def cbody(k, carry):
            idx = dst_v[k // (IDX_W // L), pl.ds((k % (IDX_W // L)) * L, L)]
            plsc.addupdate_scatter(cnt_v, [idx], ones)
            return carry
        lax.fori_loop(0, NB * IDX_W // L, cbody, 0)
        pltpu.sync_copy(cnt_v.at[pl.ds(0, N)], cnt_out.at[s])

    plsc.subcore_barrier()
    _segsum_body(xT_hbm, agg_sh, src_v, dst_v, rb0, rb1, sem0, sem1, c)
    plsc.subcore_barrier()
    pltpu.sync_copy(agg_sh.at[pl.ds(s * OROWS, OROWS)],
                    agg_out.at[c].at[pl.ds(s * OROWS, OROWS)])


@functools.partial(
    pl.kernel,
    out_type=jax.ShapeDtypeStruct((4, N, 128), jnp.float32),
    mesh=plsc.VectorSubcoreMesh(**_MESH),
    scratch_types=(
        pltpu.VMEM((NB, IDX_W), jnp.int32),
        pltpu.VMEM((NB, IDX_W), jnp.int32),
        pltpu.VMEM((IDX_W, 128), jnp.float32),
        pltpu.VMEM((IDX_W, 128), jnp.float32),
        pltpu.VMEM_SHARED((N_PAD, 128), jnp.float32),
        pltpu.SemaphoreType.DMA,
        pltpu.SemaphoreType.DMA,
    ),
)
def _sc_agg2(h1T_hbm, srcp_hbm, dstp_hbm, zeros_hbm, agg_out,
             src_v, dst_v, rb0, rb1, agg_sh, sem0, sem1):
    c = lax.axis_index("c")
    s = lax.axis_index("s")

    pltpu.sync_copy(srcp_hbm.at[s], src_v)
    pltpu.sync_copy(dstp_hbm.at[s], dst_v)

    for q in range(2):
        chunk = c * 2 + q
        pltpu.sync_copy(zeros_hbm.at[pl.ds(s * ZROWS, ZROWS)],
                        agg_sh.at[pl.ds(s * ZROWS, ZROWS)])
        plsc.subcore_barrier()
        _segsum_body(h1T_hbm, agg_sh, src_v, dst_v, rb0, rb1, sem0, sem1,
                     chunk)
        plsc.subcore_barrier()
        pltpu.sync_copy(agg_sh.at[pl.ds(s * OROWS, OROWS)],
                        agg_out.at[chunk].at[pl.ds(s * OROWS, OROWS)])
        plsc.subcore_barrier()


@functools.partial(
    pl.kernel,
    out_type=jax.ShapeDtypeStruct((E_PAD2,), jnp.float32),
    mesh=plsc.VectorSubcoreMesh(**_MESH),
    scratch_types=(
        pltpu.VMEM((2 * N,), jnp.float32),
        pltpu.VMEM((EPW,), jnp.int32),
        pltpu.VMEM((EPW,), jnp.int32),
        pltpu.VMEM((EPW,), jnp.float32),
    ),
)
def _sc_edge(ab_hbm, src_hbm, dst_hbm, out_hbm, ab_v, src_v, dst_v, out_v):
    c = lax.axis_index("c")
    s = lax.axis_index("s")
    w = s * NC + c

    pltpu.sync_copy(ab_hbm, ab_v)
    pltpu.sync_copy(src_hbm.at[pl.ds(w * EPW, EPW)], src_v)
    pltpu.sync_copy(dst_hbm.at[pl.ds(w * EPW, EPW)], dst_v)

    def body(i, carry):
        ss = src_v[pl.ds(i * L, L)]
        dd = dst_v[pl.ds(i * L, L)]
        va = plsc.load_gather(ab_v, [ss * 2])
        vb = plsc.load_gather(ab_v, [dd * 2 + 1])
        out_v[pl.ds(i * L, L)] = va + vb
        return carry

    lax.fori_loop(0, EPW // L, body, 0)
    pltpu.sync_copy(out_v, out_hbm.at[pl.ds(w * EPW, EPW)])


def _tc_prep_kernel(x_ref, w1r_ref, b1_ref, t1_ref, xT_ref):
    c = pl.program_id(1)
    part = jnp.dot(x_ref[...], w1r_ref[...],
                   preferred_element_type=jnp.float32)

    @pl.when(c == 0)
    def _():
        t1_ref[...] = part + b1_ref[...]

    @pl.when(c != 0)
    def _():
        t1_ref[...] += part

    xT_ref[0] = x_ref[...]


def _tc_prep(x, W1r, b1):
    return pl.pallas_call(
        _tc_prep_kernel,
        grid=(G, 2),
        in_specs=[
            pl.BlockSpec((R, 128), lambda r, c: (r, c)),
            pl.BlockSpec((128, D_HID), lambda r, c: (c, 0)),
            pl.BlockSpec((1, D_HID), lambda r, c: (0, 0)),
        ],
        out_specs=[
            pl.BlockSpec((R, D_HID), lambda r, c: (r, 0)),
            pl.BlockSpec((1, R, 128), lambda r, c: (c, r, 0)),
        ],
        out_shape=[
            jax.ShapeDtypeStruct((N, D_HID), jnp.float32),
            jax.ShapeDtypeStruct((2, N, 128), jnp.float32),
        ],
    )(x, W1r, b1.reshape(1, D_HID))


def _tc_layer1_kernel(agg1_ref, cntp_ref, t1_ref, w1l_ref, w2r_ref, b2_ref,
                      t2_ref, h1T_ref):
    cnt = jnp.sum(cntp_ref[...], axis=0)
    inv = 1.0 / jnp.maximum(cnt, 1.0)
    h = t1_ref[...]
    for q in range(2):
        m = agg1_ref[q] * inv[:, None]
        h = h + jnp.dot(m, w1l_ref[q * 128:(q + 1) * 128, :],
                        preferred_element_type=jnp.float32)
    h = jnp.maximum(h, 0.0)
    t2_ref[...] = jnp.dot(h, w2r_ref[...],
                          preferred_element_type=jnp.float32) + b2_ref[...]
    for q in range(4):
        h1T_ref[q] = h[:, q * 128:(q + 1) * 128]


def _tc_layer1(agg1, cntp, t1, W1l, W2r, b2):
    return pl.pallas_call(
        _tc_layer1_kernel,
        grid=(G,),
        in_specs=[
            pl.BlockSpec((2, R, 128), lambda r: (0, r, 0)),
            pl.BlockSpec((NS, R), lambda r: (0, r)),
            pl.BlockSpec((R, D_HID), lambda r: (r, 0)),
            pl.BlockSpec((D_IN, D_HID), lambda r: (0, 0)),
            pl.BlockSpec((D_HID, D_HID), lambda r: (0, 0)),
            pl.BlockSpec((1, D_HID), lambda r: (0, 0)),
        ],
        out_specs=[
            pl.BlockSpec((R, D_HID), lambda r: (r, 0)),
            pl.BlockSpec((4, R, 128), lambda r: (0, r, 0)),
        ],
        out_shape=[
            jax.ShapeDtypeStruct((N, D_HID), jnp.float32),
            jax.ShapeDtypeStruct((4, N, 128), jnp.float32),
        ],
    )(agg1, cntp, t1, W1l, W2r, b2.reshape(1, D_HID))


def _tc_layer2_kernel(agg2_ref, cntp_ref, t2_ref, w2l_ref, we2_ref, bea_ref,
                      ab_ref):
    cnt = jnp.sum(cntp_ref[...], axis=0)
    inv = 1.0 / jnp.maximum(cnt, 1.0)
    h = t2_ref[...]
    for q in range(4):
        m = agg2_ref[q] * inv[:, None]
        h = h + jnp.dot(m, w2l_ref[q * 128:(q + 1) * 128, :],
                        preferred_element_type=jnp.float32)
    ab_ref[...] = jnp.dot(h, we2_ref[...],
                          preferred_element_type=jnp.float32) + bea_ref[...]


def _tc_layer2(agg2, cntp, t2, W2l, we2, bea):
    return pl.pallas_call(
        _tc_layer2_kernel,
        grid=(G,),
        in_specs=[
            pl.BlockSpec((4, R, 128), lambda r: (0, r, 0)),
            pl.BlockSpec((NS, R), lambda r: (0, r)),
            pl.BlockSpec((R, D_HID), lambda r: (r, 0)),
            pl.BlockSpec((D_HID, D_HID), lambda r: (0, 0)),
            pl.BlockSpec((D_HID, 2), lambda r: (0, 0)),
            pl.BlockSpec((1, 2), lambda r: (0, 0)),
        ],
        out_specs=pl.BlockSpec((R, 2), lambda r: (r, 0)),
        out_shape=jax.ShapeDtypeStruct((N, 2), jnp.float32),
    )(agg2, cntp, t2, W2l, we2, bea)


def kernel(x, edge_index, W1l, b1, W1r, W2l, b2, W2r, We, be):
    src = edge_index[0]
    dst = edge_index[1]

    srcp = jnp.concatenate(
        [src, jnp.zeros((E_PAD1 - E,), jnp.int32)]).reshape(NS, NB, IDX_W)
    dstp = jnp.concatenate(
        [dst, jnp.full((E_PAD1 - E,), N, jnp.int32)]).reshape(NS, NB, IDX_W)
    zeros = jnp.zeros((N_PAD, 128), jnp.float32)

    src2 = jnp.concatenate([src, jnp.zeros((E_PAD2 - E,), jnp.int32)])
    dst2 = jnp.concatenate([dst, jnp.zeros((E_PAD2 - E,), jnp.int32)])

    we2 = jnp.concatenate([We[:D_HID], We[D_HID:]], axis=1)
    bea = jnp.concatenate([be.reshape(1, 1),
                           jnp.zeros((1, 1), jnp.float32)], axis=1)

    t1, xT = _tc_prep(x, W1r, b1)
    agg1, cntp = _sc_agg1(xT, srcp, dstp, zeros)
    t2, h1T = _tc_layer1(agg1, cntp, t1, W1l, W2r, b2)
    agg2 = _sc_agg2(h1T, srcp, dstp, zeros)
    ab = _tc_layer2(agg2, cntp, t2, W2l, we2, bea)
    out = _sc_edge(ab.reshape(2 * N), src2, dst2)
    return out[:E].reshape(E, 1)


# speedup vs baseline: 2.9771x; 2.9771x over previous
import functools

import jax
import jax.numpy as jnp
from jax import lax
from jax.experimental import pallas as pl
from jax.experimental.pallas import tpu as pltpu
from jax.experimental.pallas import tpu_sc as plsc

N = 10000
E = 160000
D_IN = 256
D_HID = 512

NC = 2
NS = 16
L = 16

CW = 64
NCH1 = D_IN // CW
NCH2 = D_HID // CW
IDX_W = 128
NB = 80
EPT = NB * IDX_W
E_PAD1 = NS * EPT
N_PAD = 10240
ZROWS = N_PAD // NS

EPW = 5008
E_PAD2 = NC * NS * EPW

R = 2000
G = N // R

_MESH = dict(core_axis_name="c", subcore_axis_name="s")
_PARAMS = pltpu.CompilerParams(needs_layout_passes=False,
                               use_tc_tiling_on_sc=False)


def _segsum_body(table_hbm, agg_sh, src_v, dst_v, rb0, rb1, sem0,
                 sem1, chunk):
    tbl = table_hbm.at[chunk]

    def start(b, buf, sem):
        pltpu.async_copy(tbl.at[src_v.at[b]], buf, sem)

    def wait(buf, sem):
        pltpu.make_async_copy(tbl.at[src_v.at[0]], buf, sem).wait()

    start(0, rb0, sem0)
    start(1, rb1, sem1)

    def body(k, carry):
        b0 = 2 * k
        wait(rb0, sem0)
        pltpu.sync_copy(rb0, agg_sh.at[dst_v.at[b0]], add=True)

        @pl.when(k < NB // 2 - 1)
        def _():
            start(b0 + 2, rb0, sem0)

        wait(rb1, sem1)
        pltpu.sync_copy(rb1, agg_sh.at[dst_v.at[b0 + 1]], add=True)

        @pl.when(k < NB // 2 - 1)
        def _():
            start(b0 + 3, rb1, sem1)

        return carry

    lax.fori_loop(0, NB // 2, body, 0)


@functools.partial(
    pl.kernel,
    out_type=(
        jax.ShapeDtypeStruct((NCH1, N_PAD, CW), jnp.float32),
        jax.ShapeDtypeStruct((N,), jnp.float32),
    ),
    mesh=plsc.VectorSubcoreMesh(**_MESH),
    compiler_params=_PARAMS,
    scratch_types=(
        pltpu.VMEM((NB, IDX_W), jnp.int32),
        pltpu.VMEM((NB, IDX_W), jnp.int32),
        pltpu.VMEM((IDX_W, CW), jnp.float32),
        pltpu.VMEM((IDX_W, CW), jnp.float32),
        pltpu.VMEM((IDX_W, L), jnp.float32),
        pltpu.VMEM((ZROWS, L), jnp.float32),
        pltpu.VMEM((ZROWS,), jnp.float32),
        pltpu.VMEM_SHARED((N_PAD, CW), jnp.float32),
        pltpu.VMEM_SHARED((N_PAD, L), jnp.float32),
        pltpu.SemaphoreType.DMA,
        pltpu.SemaphoreType.DMA,
        pltpu.SemaphoreType.DMA,
    ),
)
def _sc_agg1(xT_hbm, srcp_hbm, dstp_hbm, zeros_hbm, zeros16_hbm, agg_out,
             inv_out, src_v, dst_v, rb0, rb1, ones_v, cntc_v, inv_v, agg_sh,
             cnt_sh, sem0, sem1, sem2):
    c = lax.axis_index("c")
    s = lax.axis_index("s")

    pltpu.sync_copy(srcp_hbm.at[s], src_v)
    pltpu.sync_copy(dstp_hbm.at[s], dst_v)

    @pl.when(c == 0)
    def _():
        pltpu.sync_copy(zeros16_hbm.at[pl.ds(s * ZROWS, ZROWS)],
                        cnt_sh.at[pl.ds(s * ZROWS, ZROWS)])

        def obody(i, carry):
            ones_v[i] = jnp.ones((L,), jnp.float32)
            return carry
        lax.fori_loop(0, IDX_W, obody, 0)

    fired = 0
    for q in range(NCH1 // NC):
        chunk = c * (NCH1 // NC) + q
        pltpu.sync_copy(zeros_hbm.at[pl.ds(s * ZROWS, ZROWS)],
                        agg_sh.at[pl.ds(s * ZROWS, ZROWS)])
        plsc.subcore_barrier()

        if not fired:
            fired = 1

            @pl.when(c == 0)
            def _():
                def fbody(b, carry):
                    pltpu.async_copy(ones_v, cnt_sh.at[dst_v.at[b]], sem2,
                                     add=True)
                    return carry
                lax.fori_loop(0, NB, fbody, 0)

        _segsum_body(xT_hbm, agg_sh, src_v, dst_v, rb0, rb1, sem0, sem1,
                     chunk)
        plsc.subcore_barrier()
        pltpu.sync_copy(agg_sh.at[pl.ds(s * ZROWS, ZROWS)],
                        agg_out.at[chunk].at[pl.ds(s * ZROWS, ZROWS)])

    @pl.when(c == 0)
    def _():
        def dbody(b, carry):
            pltpu.make_async_copy(ones_v, cnt_sh.at[dst_v.at[0]],
                                  sem2).wait()
            return carry
        lax.fori_loop(0, NB, dbody, 0)

    plsc.subcore_barrier()

    @pl.when(c == 0)
    def _():
        pltpu.sync_copy(cnt_sh.at[pl.ds(s * ZROWS, ZROWS)], cntc_v)
        col0 = jnp.zeros((L,), jnp.int32)
        rows = lax.iota(jnp.int32, L)

        def rbody(t, carry):
            cnt = plsc.load_gather(cntc_v, [t * L + rows, col0])
            inv_v[pl.ds(t * L, L)] = 1.0 / jnp.maximum(cnt, 1.0)
            return carry
        lax.fori_loop(0, ZROWS // L, rbody, 0)

        @pl.when(s < NS - 1)
        def _():
            pltpu.sync_copy(inv_v, inv_out.at[pl.ds(s * ZROWS, ZROWS)])

        @pl.when(s == NS - 1)
        def _():
            pltpu.sync_copy(inv_v.at[pl.ds(0, N - (NS - 1) * ZROWS)],
                            inv_out.at[pl.ds((NS - 1) * ZROWS,
                                             N - (NS - 1) * ZROWS)])


@functools.partial(
    pl.kernel,
    out_type=jax.ShapeDtypeStruct((NCH2, N_PAD, CW), jnp.float32),
    mesh=plsc.VectorSubcoreMesh(**_MESH),
    compiler_params=_PARAMS,
    scratch_types=(
        pltpu.VMEM((NB, IDX_W), jnp.int32),
        pltpu.VMEM((NB, IDX_W), jnp.int32),
        pltpu.VMEM((IDX_W, CW), jnp.float32),
        pltpu.VMEM((IDX_W, CW), jnp.float32),
        pltpu.VMEM_SHARED((N_PAD, CW), jnp.float32),
        pltpu.SemaphoreType.DMA,
        pltpu.SemaphoreType.DMA,
    ),
)
def _sc_agg2(h1T_hbm, srcp_hbm, dstp_hbm, zeros_hbm, agg_out,
             src_v, dst_v, rb0, rb1, agg_sh, sem0, sem1):
    c = lax.axis_index("c")
    s = lax.axis_index("s")

    pltpu.sync_copy(srcp_hbm.at[s], src_v)
    pltpu.sync_copy(dstp_hbm.at[s], dst_v)

    for q in range(NCH2 // NC):
        chunk = c * (NCH2 // NC) + q
        pltpu.sync_copy(zeros_hbm.at[pl.ds(s * ZROWS, ZROWS)],
                        agg_sh.at[pl.ds(s * ZROWS, ZROWS)])
        plsc.subcore_barrier()
        _segsum_body(h1T_hbm, agg_sh, src_v, dst_v, rb0, rb1, sem0, sem1,
                     chunk)
        plsc.subcore_barrier()
        pltpu.sync_copy(agg_sh.at[pl.ds(s * ZROWS, ZROWS)],
                        agg_out.at[chunk].at[pl.ds(s * ZROWS, ZROWS)])


@functools.partial(
    pl.kernel,
    out_type=jax.ShapeDtypeStruct((E_PAD2,), jnp.float32),
    mesh=plsc.VectorSubcoreMesh(**_MESH),
    compiler_params=_PARAMS,
    scratch_types=(
        pltpu.VMEM((2 * N,), jnp.float32),
        pltpu.VMEM((EPW,), jnp.int32),
        pltpu.VMEM((EPW,), jnp.int32),
        pltpu.VMEM((EPW,), jnp.float32),
    ),
)
def _sc_edge(ab_hbm, src_hbm, dst_hbm, out_hbm, ab_v, src_v, dst_v, out_v):
    c = lax.axis_index("c")
    s = lax.axis_index("s")
    w = s * NC + c

    pltpu.sync_copy(ab_hbm, ab_v)
    pltpu.sync_copy(src_hbm.at[pl.ds(w * EPW, EPW)], src_v)
    pltpu.sync_copy(dst_hbm.at[pl.ds(w * EPW, EPW)], dst_v)

    def body(i, carry):
        ss = src_v[pl.ds(i * L, L)]
        dd = dst_v[pl.ds(i * L, L)]
        va = plsc.load_gather(ab_v, [ss * 2])
        vb = plsc.load_gather(ab_v, [dd * 2 + 1])
        out_v[pl.ds(i * L, L)] = va + vb
        return carry

    lax.fori_loop(0, EPW // L, body, 0)
    pltpu.sync_copy(out_v, out_hbm.at[pl.ds(w * EPW, EPW)])


def _tc_prep_kernel(x_ref, w1r_ref, b1_ref, t1_ref, xT_ref):
    x = x_ref[...]
    t1_ref[...] = jnp.dot(x, w1r_ref[...],
                          preferred_element_type=jnp.float32,
                          precision=jax.lax.Precision.HIGHEST) + b1_ref[...]
    for q in range(NCH1):
        xT_ref[q] = x[:, q * CW:(q + 1) * CW]


def _tc_prep(x, W1r, b1):
    return pl.pallas_call(
        _tc_prep_kernel,
        grid=(G,),
        in_specs=[
            pl.BlockSpec((R, D_IN), lambda r: (r, 0)),
            pl.BlockSpec((D_IN, D_HID), lambda r: (0, 0)),
            pl.BlockSpec((1, D_HID), lambda r: (0, 0)),
        ],
        out_specs=[
            pl.BlockSpec((R, D_HID), lambda r: (r, 0)),
            pl.BlockSpec((NCH1, R, CW), lambda r: (0, r, 0)),
        ],
        out_shape=[
            jax.ShapeDtypeStruct((N, D_HID), jnp.float32),
            jax.ShapeDtypeStruct((NCH1, N, CW), jnp.float32),
        ],
    )(x, W1r, b1.reshape(1, D_HID))


def _tc_layer1_kernel(agg1_ref, inv_ref, t1_ref, w1l_ref, w2r_ref, b2_ref,
                      t2_ref, h1T_ref):
    inv = inv_ref[...]
    h = t1_ref[...]
    for q in range(NCH1):
        m = agg1_ref[q] * inv
        h = h + jnp.dot(m, w1l_ref[q * CW:(q + 1) * CW, :],
                        preferred_element_type=jnp.float32,
                          precision=jax.lax.Precision.HIGHEST)
    h = jnp.maximum(h, 0.0)
    t2_ref[...] = jnp.dot(h, w2r_ref[...],
                          preferred_element_type=jnp.float32,
                          precision=jax.lax.Precision.HIGHEST) + b2_ref[...]
    for q in range(NCH2):
        h1T_ref[q] = h[:, q * CW:(q + 1) * CW]


def _tc_layer1(agg1, inv, t1, W1l, W2r, b2):
    return pl.pallas_call(
        _tc_layer1_kernel,
        grid=(G,),
        in_specs=[
            pl.BlockSpec((NCH1, R, CW), lambda r: (0, r, 0)),
            pl.BlockSpec((R, 1), lambda r: (r, 0)),
            pl.BlockSpec((R, D_HID), lambda r: (r, 0)),
            pl.BlockSpec((D_IN, D_HID), lambda r: (0, 0)),
            pl.BlockSpec((D_HID, D_HID), lambda r: (0, 0)),
            pl.BlockSpec((1, D_HID), lambda r: (0, 0)),
        ],
        out_specs=[
            pl.BlockSpec((R, D_HID), lambda r: (r, 0)),
            pl.BlockSpec((NCH2, R, CW), lambda r: (0, r, 0)),
        ],
        out_shape=[
            jax.ShapeDtypeStruct((N, D_HID), jnp.float32),
            jax.ShapeDtypeStruct((NCH2, N, CW), jnp.float32),
        ],
    )(agg1, inv, t1, W1l, W2r, b2.reshape(1, D_HID))


def _tc_layer2_kernel(agg2_ref, inv_ref, t2_ref, w2l_ref, we2_ref, bea_ref,
                      ab_ref):
    inv = inv_ref[...]
    h = t2_ref[...]
    for q in range(NCH2):
        m = agg2_ref[q] * inv
        h = h + jnp.dot(m, w2l_ref[q * CW:(q + 1) * CW, :],
                        preferred_element_type=jnp.float32,
                          precision=jax.lax.Precision.HIGHEST)
    ab_ref[...] = jnp.dot(h, we2_ref[...],
                          preferred_element_type=jnp.float32,
                          precision=jax.lax.Precision.HIGHEST) + bea_ref[...]


def _tc_layer2(agg2, inv, t2, W2l, we2, bea):
    return pl.pallas_call(
        _tc_layer2_kernel,
        grid=(G,),
        in_specs=[
            pl.BlockSpec((NCH2, R, CW), lambda r: (0, r, 0)),
            pl.BlockSpec((R, 1), lambda r: (r, 0)),
            pl.BlockSpec((R, D_HID), lambda r: (r, 0)),
            pl.BlockSpec((D_HID, D_HID), lambda r: (0, 0)),
            pl.BlockSpec((D_HID, 2), lambda r: (0, 0)),
            pl.BlockSpec((1, 2), lambda r: (0, 0)),
        ],
        out_specs=pl.BlockSpec((R, 2), lambda r: (r, 0)),
        out_shape=jax.ShapeDtypeStruct((N, 2), jnp.float32),
    )(agg2, inv, t2, W2l, we2, bea)


def kernel(x, edge_index, W1l, b1, W1r, W2l, b2, W2r, We, be):
    src = edge_index[0]
    dst = edge_index[1]

    srcp = jnp.concatenate(
        [src, jnp.zeros((E_PAD1 - E,), jnp.int32)]).reshape(NS, NB, IDX_W)
    dstp = jnp.concatenate(
        [dst, jnp.full((E_PAD1 - E,), N, jnp.int32)]).reshape(NS, NB, IDX_W)
    zeros = jnp.zeros((N_PAD, CW), jnp.float32)
    zeros16 = jnp.zeros((N_PAD, L), jnp.float32)

    src2 = jnp.concatenate([src, jnp.zeros((E_PAD2 - E,), jnp.int32)])
    dst2 = jnp.concatenate([dst, jnp.zeros((E_PAD2 - E,), jnp.int32)])

    we2 = jnp.concatenate([We[:D_HID], We[D_HID:]], axis=1)
    bea = jnp.concatenate([be.reshape(1, 1),
                           jnp.zeros((1, 1), jnp.float32)], axis=1)

    t1, xT = _tc_prep(x, W1r, b1)
    agg1, inv = _sc_agg1(xT, srcp, dstp, zeros, zeros16)
    inv = inv.reshape(N, 1)
    t2, h1T = _tc_layer1(agg1, inv, t1, W1l, W2r, b2)
    agg2 = _sc_agg2(h1T, srcp, dstp, zeros)
    ab = _tc_layer2(agg2, inv, t2, W2l, we2, bea)
    out = _sc_edge(ab.reshape(2 * N), src2, dst2)
    return out[:E].reshape(E, 1)
